# baseline (speedup 1.0000x reference)
import functools

import jax
import jax.numpy as jnp
from jax import lax
from jax.experimental import pallas as pl
from jax.experimental.pallas import tpu as pltpu
from jax.experimental.pallas import tpu_sc as plsc

_K = 10
_NU = 1000000


def _sc_call(users, questions, alpha1, beta_b, Qp):
    B = questions.shape[0]
    NQP = Qp.shape[1]
    info = plsc.get_sparse_core_info()
    NC, NS, L = info.num_cores, info.num_subcores, info.num_lanes
    NW = NC * NS
    b_per_w = B // NW
    n_words = b_per_w * _K
    J = n_words // 128

    mesh = plsc.VectorSubcoreMesh(core_axis_name="c", subcore_axis_name="s")

    @functools.partial(
        pl.kernel,
        mesh=mesh,
        out_type=jax.ShapeDtypeStruct((B,), jnp.float32),
        compiler_params=pltpu.CompilerParams(
            needs_layout_passes=False, use_tc_tiling_on_sc=False),
        scratch_types=[
            pltpu.VMEM((b_per_w,), jnp.int32),
            pltpu.VMEM((b_per_w,), jnp.int32),
            pltpu.VMEM((J, 128), jnp.int32),
            pltpu.VMEM((n_words,), jnp.float32),
            pltpu.VMEM((_K, NQP), jnp.float32),
            pltpu.VMEM((_K, 16), jnp.float32),
            pltpu.VMEM((b_per_w,), jnp.float32),
            pltpu.SemaphoreType.DMA,
        ],
    )
    def k(users_ref, q_ref, alpha_ref, beta_ref, Q_ref, out_ref,
          u_v, q_v, widx_v, rows_v, Q_v, beta_v, out_v, sem):
        wid = lax.axis_index("s") * NC + lax.axis_index("c")
        base = wid * b_per_w

        pltpu.sync_copy(users_ref.at[pl.ds(base, b_per_w)], u_v)

        def build(c, carry):
            p = c * L + lax.iota(jnp.int32, L)
            ui = plsc.load_gather(u_v, [p // _K])
            wi = (p % _K) * _NU + ui
            widx_v[c // 8, pl.ds((c % 8) * L, L)] = wi
            return carry

        lax.fori_loop(0, n_words // L, build, 0)

        copies = [
            pltpu.async_copy(alpha_ref.at[widx_v.at[j]],
                             rows_v.at[pl.ds(j * 128, 128)], sem)
            for j in range(J)
        ]
        pltpu.sync_copy(q_ref.at[pl.ds(base, b_per_w)], q_v)
        pltpu.sync_copy(Q_ref, Q_v)
        pltpu.sync_copy(beta_ref, beta_v)
        for c in copies:
            c.wait()

        betas = [beta_v[kk] for kk in range(_K)]

        def body(c, carry):
            i = c * L + lax.iota(jnp.int32, L)
            qs = q_v[pl.ds(c * L, L)]
            acc = jnp.zeros((L,), jnp.float32)
            for kk in range(_K):
                a_k = plsc.load_gather(rows_v, [i * _K + kk])
                q_k = plsc.load_gather(
                    Q_v, [jnp.full((L,), kk, jnp.int32), qs])
                acc = acc + (a_k + betas[kk]) * q_k
            out_v[pl.ds(c * L, L)] = 1.0 / (1.0 + jnp.exp(-acc))
            return carry

        lax.fori_loop(0, b_per_w // L, body, 0)
        pltpu.sync_copy(out_v, out_ref.at[pl.ds(base, b_per_w)])

    return k(users, questions, alpha1, beta_b, Qp)


@jax.jit
def kernel(users, questions, alpha, beta, Q):
    alpha1 = jnp.reshape(alpha.T, (-1,))
    beta_b = jnp.broadcast_to(beta[:, None], (beta.shape[0], 16))
    Qp = jnp.pad(Q, ((0, 0), (0, (-Q.shape[1]) % 8)))
    return _sc_call(users, questions, alpha1, beta_b, Qp)


# speedup vs baseline: 1.3850x; 1.3850x over previous
import functools

import jax
import jax.numpy as jnp
from jax import lax
from jax.experimental import pallas as pl
from jax.experimental.pallas import tpu as pltpu
from jax.experimental.pallas import tpu_sc as plsc

_K = 10
_NU = 1000000


def _sc_call(users, questions, alpha1, beta_b, Qp):
    B = questions.shape[0]
    NQP = Qp.shape[1]
    info = plsc.get_sparse_core_info()
    NC, NS, L = info.num_cores, info.num_subcores, info.num_lanes
    NW = NC * NS
    b_per_w = B // NW
    n_words = b_per_w * _K
    J = n_words // 128

    mesh = plsc.VectorSubcoreMesh(core_axis_name="c", subcore_axis_name="s")

    @functools.partial(
        pl.kernel,
        mesh=mesh,
        out_type=jax.ShapeDtypeStruct((B,), jnp.float32),
        compiler_params=pltpu.CompilerParams(
            needs_layout_passes=False, use_tc_tiling_on_sc=False),
        scratch_types=[
            pltpu.VMEM((b_per_w,), jnp.int32),
            pltpu.VMEM((b_per_w,), jnp.int32),
            pltpu.VMEM((J, 128), jnp.int32),
            pltpu.VMEM((n_words,), jnp.float32),
            pltpu.VMEM((_K, NQP), jnp.float32),
            pltpu.VMEM((_K, 16), jnp.float32),
            pltpu.VMEM((b_per_w,), jnp.float32),
            pltpu.SemaphoreType.DMA,
        ],
    )
    def k(users_ref, q_ref, alpha_ref, beta_ref, Q_ref, out_ref,
          u_v, q_v, widx_v, rows_v, Q_v, beta_v, out_v, sem):
        wid = lax.axis_index("s") * NC + lax.axis_index("c")
        base = wid * b_per_w

        pltpu.sync_copy(users_ref.at[pl.ds(base, b_per_w)], u_v)

        def build(c, carry):
            p = c * L + lax.iota(jnp.int32, L)
            ui = plsc.load_gather(u_v, [p // _K])
            wi = ui * _K + p % _K
            widx_v[c // 8, pl.ds((c % 8) * L, L)] = wi
            return carry

        lax.fori_loop(0, n_words // L, build, 0)

        copies = [
            pltpu.async_copy(alpha_ref.at[widx_v.at[j]],
                             rows_v.at[pl.ds(j * 128, 128)], sem)
            for j in range(J)
        ]
        pltpu.sync_copy(q_ref.at[pl.ds(base, b_per_w)], q_v)
        pltpu.sync_copy(Q_ref, Q_v)
        pltpu.sync_copy(beta_ref, beta_v)
        for c in copies:
            c.wait()

        betas = [beta_v[kk] for kk in range(_K)]

        def body(c, carry):
            i = c * L + lax.iota(jnp.int32, L)
            qs = q_v[pl.ds(c * L, L)]
            acc = jnp.zeros((L,), jnp.float32)
            for kk in range(_K):
                a_k = plsc.load_gather(rows_v, [i * _K + kk])
                q_k = plsc.load_gather(
                    Q_v, [jnp.full((L,), kk, jnp.int32), qs])
                acc = acc + (a_k + betas[kk]) * q_k
            out_v[pl.ds(c * L, L)] = 1.0 / (1.0 + jnp.exp(-acc))
            return carry

        lax.fori_loop(0, b_per_w // L, body, 0)
        pltpu.sync_copy(out_v, out_ref.at[pl.ds(base, b_per_w)])

    return k(users, questions, alpha1, beta_b, Qp)


@jax.jit
def kernel(users, questions, alpha, beta, Q):
    alpha1 = jnp.reshape(alpha, (-1,)) * 1.0 + 0.0
    beta_b = jnp.broadcast_to(beta[:, None], (beta.shape[0], 16))
    Qp = jnp.pad(Q, ((0, 0), (0, (-Q.shape[1]) % 8)))
    return _sc_call(users, questions, alpha1, beta_b, Qp)
